# baseline (speedup 1.0000x reference)
import jax
import jax.numpy as jnp
from jax.experimental import pallas as pl
from jax.experimental.pallas import tpu as pltpu

_NUM_SAMPLES = 16


def _copy_frame(idx_ref, x_ref, o_ref):
    del idx_ref
    o_ref[...] = x_ref[...]


@jax.jit
def kernel(x):
    b, t, c, h, w = x.shape
    n = c * h * w
    rows = n // 128
    idx = jnp.linspace(0.0, float(t - 1), _NUM_SAMPLES).astype(jnp.int32)
    xr = x.reshape(b, t, rows, 128)
    out = pl.pallas_call(
        _copy_frame,
        grid_spec=pltpu.PrefetchScalarGridSpec(
            num_scalar_prefetch=1,
            grid=(b, _NUM_SAMPLES),
            in_specs=[
                pl.BlockSpec((1, 1, rows, 128),
                             lambda i, s, idx_ref: (i, idx_ref[s], 0, 0)),
            ],
            out_specs=pl.BlockSpec((1, 1, rows, 128),
                                   lambda i, s, idx_ref: (i, s, 0, 0)),
        ),
        out_shape=jax.ShapeDtypeStruct((b, _NUM_SAMPLES, rows, 128), x.dtype),
    )(idx, xr)
    return out.reshape(b, _NUM_SAMPLES, c, h, w)


# speedup vs baseline: 5.7262x; 5.7262x over previous
import jax
import jax.numpy as jnp
from jax.experimental import pallas as pl
from jax.experimental.pallas import tpu as pltpu

_NUM_SAMPLES = 16


def _copy_frame(idx_ref, x_ref, o_ref):
    del idx_ref
    o_ref[...] = x_ref[...]


@jax.jit
def kernel(x):
    b, t, c, h, w = x.shape
    idx = jnp.linspace(0.0, float(t - 1), _NUM_SAMPLES).astype(jnp.int32)
    out = pl.pallas_call(
        _copy_frame,
        grid_spec=pltpu.PrefetchScalarGridSpec(
            num_scalar_prefetch=1,
            grid=(b, _NUM_SAMPLES),
            in_specs=[
                pl.BlockSpec((1, 1, c, h, w),
                             lambda i, s, idx_ref: (i, idx_ref[s], 0, 0, 0)),
            ],
            out_specs=pl.BlockSpec((1, 1, c, h, w),
                                   lambda i, s, idx_ref: (i, s, 0, 0, 0)),
        ),
        out_shape=jax.ShapeDtypeStruct((b, _NUM_SAMPLES, c, h, w), x.dtype),
    )(idx, x)
    return out
